# Initial kernel scaffold; baseline (speedup 1.0000x reference)
import jax
import jax.numpy as jnp
from jax.experimental import pallas as pl


def kernel(x, edge_index, W1, b1, W2, b2, W3, b3):
    raise NotImplementedError("write your pallas kernel here")


# speedup vs baseline: 5.3518x; 5.3518x over previous
import functools

import jax
import jax.numpy as jnp
from jax import lax
from jax.experimental import pallas as pl
from jax.experimental.pallas import tpu as pltpu
from jax.experimental.pallas import tpu_sc as plsc

N = 10000
E = 320000
D = 128
NCLS = 40

NC = 2
NS = 16
K = 128
CHT = 160
E_PAD = NS * CHT * K
R_PAD = 10240
HALF = R_PAD // NC
ACC_R = HALF + K
ZR = 64
RPT = HALF // NS


def _deg_body(colsr_hbm, out_hbm, col_v, ones_v, zbuf, acc):
    c = lax.axis_index("c")
    s = lax.axis_index("s")
    wid = c * NS + s

    def _fill(r, _):
        def _fj(j, _):
            ones_v[r, pl.ds(j * 16, 16)] = jnp.ones((16,), jnp.float32)
            return 0
        return lax.fori_loop(0, D // 16, _fj, 0)
    lax.fori_loop(0, K, _fill, 0)

    def _zr(r, _):
        def _zj(j, _):
            zbuf[r, pl.ds(j * 16, 16)] = jnp.zeros((16,), jnp.float32)
            return 0
        return lax.fori_loop(0, D // 16, _zj, 0)
    lax.fori_loop(0, ZR, _zr, 0)

    def _zc(k, _):
        pltpu.sync_copy(zbuf, acc.at[pl.ds(s * RPT + k * ZR, ZR)])
        return 0
    lax.fori_loop(0, RPT // ZR, _zc, 0)

    pltpu.sync_copy(colsr_hbm.at[wid], col_v)
    plsc.subcore_barrier()

    def _step(i, _):
        pltpu.sync_copy(ones_v, acc.at[col_v.at[i]], add=True)
        return 0
    lax.fori_loop(0, CHT, _step, 0)

    plsc.subcore_barrier()
    pltpu.sync_copy(acc.at[pl.ds(s * RPT, RPT)],
                    out_hbm.at[c, pl.ds(s * RPT, RPT)])


def _scatter_body(y_hbm, rows_hbm, colsr_hbm, out_hbm,
                  row_v, col_v, gbuf, zbuf, acc, sem):
    c = lax.axis_index("c")
    s = lax.axis_index("s")
    wid = c * NS + s

    def _zr(r, _):
        def _zj(j, _):
            zbuf[r, pl.ds(j * 16, 16)] = jnp.zeros((16,), jnp.float32)
            return 0
        return lax.fori_loop(0, D // 16, _zj, 0)
    lax.fori_loop(0, ZR, _zr, 0)

    def _zc(k, _):
        pltpu.sync_copy(zbuf, acc.at[pl.ds(s * RPT + k * ZR, ZR)])
        return 0
    lax.fori_loop(0, RPT // ZR, _zc, 0)

    pltpu.sync_copy(rows_hbm.at[s], row_v)
    pltpu.sync_copy(colsr_hbm.at[wid], col_v)
    plsc.subcore_barrier()

    def _step(i, _):
        b = lax.rem(i, 2)
        desc = pltpu.async_copy(y_hbm.at[row_v.at[i]], gbuf.at[b], sem)

        @pl.when(i > 0)
        def _():
            pltpu.sync_copy(gbuf.at[1 - b], acc.at[col_v.at[i - 1]], add=True)

        desc.wait()
        return 0
    lax.fori_loop(0, CHT, _step, 0)
    pltpu.sync_copy(gbuf.at[(CHT - 1) % 2], acc.at[col_v.at[CHT - 1]],
                    add=True)

    plsc.subcore_barrier()
    pltpu.sync_copy(acc.at[pl.ds(s * RPT, RPT)],
                    out_hbm.at[c, pl.ds(s * RPT, RPT)])


@functools.lru_cache(maxsize=1)
def _sc_kernels():
    mesh = plsc.VectorSubcoreMesh(
        core_axis_name="c", subcore_axis_name="s",
        num_cores=NC, num_subcores=NS)
    deg_k = pl.kernel(
        _deg_body,
        out_type=jax.ShapeDtypeStruct((NC, HALF, D), jnp.float32),
        mesh=mesh,
        scratch_types=[
            pltpu.VMEM((CHT, K), jnp.int32),
            pltpu.VMEM((K, D), jnp.float32),
            pltpu.VMEM((ZR, D), jnp.float32),
            pltpu.VMEM_SHARED((ACC_R, D), jnp.float32),
        ],
    )
    scat_k = pl.kernel(
        _scatter_body,
        out_type=jax.ShapeDtypeStruct((NC, HALF, D), jnp.float32),
        mesh=mesh,
        scratch_types=[
            pltpu.VMEM((CHT, K), jnp.int32),
            pltpu.VMEM((CHT, K), jnp.int32),
            pltpu.VMEM((2, K, D), jnp.float32),
            pltpu.VMEM((ZR, D), jnp.float32),
            pltpu.VMEM_SHARED((ACC_R, D), jnp.float32),
            pltpu.SemaphoreType.DMA,
        ],
    )
    return deg_k, scat_k


def _kcols_body(cols_ref, out_ref):
    cv = cols_ref[...]
    pos = lax.broadcasted_iota(jnp.int32, (1, CHT, K), 2)
    for c in range(NC):
        loc = cv - c * HALF
        oob = (loc < 0) | (loc >= HALF)
        out_ref[c] = jnp.where(oob, HALF + pos, loc)


def _k0_body(deg_ref, dinvb_ref):
    cnt = jnp.max(deg_ref[...], axis=1, keepdims=True)
    dinv = lax.rsqrt(cnt + 1.0)
    dinvb_ref[...] = jnp.broadcast_to(dinv, (D, D))


def _k1_body(dinvb_ref, x_ref, w1_ref, y1_ref):
    xw = jnp.dot(x_ref[...], w1_ref[...], preferred_element_type=jnp.float32)
    y1_ref[...] = dinvb_ref[...] * xw


def _k2_body(acc_ref, y1_ref, dinvb_ref, w2_ref, b1_ref, y2_ref):
    agg = acc_ref[...] + y1_ref[...]
    h1 = jnp.maximum(dinvb_ref[...] * agg + b1_ref[...], 0.0)
    y2_ref[...] = dinvb_ref[...] * jnp.dot(
        h1, w2_ref[...], preferred_element_type=jnp.float32)


def _k3_body(acc_ref, y2_ref, dinvb_ref, b2_ref, w3_ref, b3_ref, out_ref):
    agg = acc_ref[...] + y2_ref[...]
    h2 = jnp.maximum(dinvb_ref[...] * agg + b2_ref[...], 0.0)
    logits = jnp.dot(h2, w3_ref[...],
                     preferred_element_type=jnp.float32) + b3_ref[...]
    m = jnp.max(logits, axis=1, keepdims=True)
    e = jnp.exp(logits - m)
    out_ref[...] = e / jnp.sum(e, axis=1, keepdims=True)


_G = R_PAD // D

_blk_rows = pl.BlockSpec((D, D), lambda i: (i, 0))
_blk_full = pl.BlockSpec((D, D), lambda i: (0, 0))
_blk_bias = pl.BlockSpec((1, D), lambda i: (0, 0))

_kcols = pl.pallas_call(
    _kcols_body,
    grid=(NS,),
    in_specs=[pl.BlockSpec((1, CHT, K), lambda i: (i, 0, 0))],
    out_specs=pl.BlockSpec((NC, 1, CHT, K), lambda i: (0, i, 0, 0)),
    out_shape=jax.ShapeDtypeStruct((NC, NS, CHT, K), jnp.int32),
)

_k0 = pl.pallas_call(
    _k0_body,
    grid=(_G,),
    in_specs=[_blk_rows],
    out_specs=_blk_rows,
    out_shape=jax.ShapeDtypeStruct((R_PAD, D), jnp.float32),
)

_k1 = pl.pallas_call(
    _k1_body,
    grid=(_G,),
    in_specs=[_blk_rows, _blk_rows, _blk_full],
    out_specs=_blk_rows,
    out_shape=jax.ShapeDtypeStruct((R_PAD, D), jnp.float32),
)

_k2 = pl.pallas_call(
    _k2_body,
    grid=(_G,),
    in_specs=[_blk_rows, _blk_rows, _blk_rows, _blk_full, _blk_bias],
    out_specs=_blk_rows,
    out_shape=jax.ShapeDtypeStruct((R_PAD, D), jnp.float32),
)

_k3 = pl.pallas_call(
    _k3_body,
    grid=(_G,),
    in_specs=[_blk_rows, _blk_rows, _blk_rows, _blk_bias, _blk_full,
              _blk_bias],
    out_specs=_blk_rows,
    out_shape=jax.ShapeDtypeStruct((R_PAD, D), jnp.float32),
)


def kernel(x, edge_index, W1, b1, W2, b2, W3, b3):
    row = edge_index[0].astype(jnp.int32)
    col = edge_index[1].astype(jnp.int32)
    pad = jnp.full((E_PAD - E,), N, jnp.int32)
    rows3 = jnp.concatenate([row, pad]).reshape(NS, CHT, K)
    cols3 = jnp.concatenate([col, pad]).reshape(NS, CHT, K)

    x_pad = jnp.concatenate(
        [x, jnp.zeros((R_PAD - N, D), jnp.float32)], axis=0)
    b1r = b1.reshape(1, D)
    b2r = b2.reshape(1, D)
    W3p = jnp.concatenate(
        [W3, jnp.zeros((D, D - NCLS), jnp.float32)], axis=1)
    b3p = jnp.concatenate(
        [b3, jnp.full((D - NCLS,), -1e30, jnp.float32)]).reshape(1, D)

    deg_kernel, scatter_kernel = _sc_kernels()
    colsr = _kcols(cols3).reshape(NC * NS, CHT, K)
    deg = deg_kernel(colsr).reshape(R_PAD, D)
    dinvb = _k0(deg)
    y1 = _k1(dinvb, x_pad, W1)
    acc1 = scatter_kernel(y1, rows3, colsr).reshape(R_PAD, D)
    y2 = _k2(acc1, y1, dinvb, W2, b1r)
    acc2 = scatter_kernel(y2, rows3, colsr).reshape(R_PAD, D)
    probs = _k3(acc2, y2, dinvb, b2r, W3p, b3p)
    return probs[:N, :NCLS]
